# Initial kernel scaffold; baseline (speedup 1.0000x reference)
import jax
import jax.numpy as jnp
from jax.experimental import pallas as pl


def kernel(inputs, pilots):
    raise NotImplementedError("write your pallas kernel here")


# speedup vs baseline: 17.5985x; 17.5985x over previous
import jax
import jax.numpy as jnp
from jax.experimental import pallas as pl

_NUM_SYM = 14
_FFT = 4096
_N = 2
_BATCH = 128
_ROW_IN = 12 * _FFT * _N
_ROW_OUT = _NUM_SYM * _FFT * _N
_SEG = _FFT * _N

_BB = 8


def _body(x_ref, p_ref, o_ref):
    o_ref[:, 0:2 * _SEG] = x_ref[:, 0:2 * _SEG]
    o_ref[:, 3 * _SEG:11 * _SEG] = x_ref[:, 2 * _SEG:10 * _SEG]
    o_ref[:, 12 * _SEG:14 * _SEG] = x_ref[:, 10 * _SEG:12 * _SEG]
    pr = jnp.repeat(p_ref[...], _N, axis=1)
    o_ref[:, 2 * _SEG:3 * _SEG] = jnp.broadcast_to(pr[0:1, :], (_BB, _SEG))
    o_ref[:, 11 * _SEG:12 * _SEG] = jnp.broadcast_to(pr[1:2, :], (_BB, _SEG))


def kernel(inputs, pilots):
    b = inputs.shape[0]
    x = inputs.reshape(b, _ROW_IN)
    p2 = pilots.reshape(2, _FFT)
    out = pl.pallas_call(
        _body,
        grid=(b // _BB,),
        in_specs=[
            pl.BlockSpec((_BB, _ROW_IN), lambda i: (i, 0)),
            pl.BlockSpec((2, _FFT), lambda i: (0, 0)),
        ],
        out_specs=pl.BlockSpec((_BB, _ROW_OUT), lambda i: (i, 0)),
        out_shape=jax.ShapeDtypeStruct((b, _ROW_OUT), inputs.dtype),
    )(x, p2)
    return out.reshape(b, 1, 1, _NUM_SYM, _FFT, _N)
